# baseline (speedup 1.0000x reference)
import functools

import jax
import jax.numpy as jnp
from jax import lax
from jax.experimental import pallas as pl
from jax.experimental.pallas import tpu as pltpu
from jax.experimental.pallas import tpu_sc as plsc

N = 100000
F = 64
C = 751
CP = 768
B = 1024
TEMP = 0.05

NC = 2
NS = 16
NW = NC * NS

CHUNK = 128
SLOT = 256
MAIN_ITERS = 6
MAIN_ROWS = NW * MAIN_ITERS * SLOT
HALF1 = MAIN_ROWS
HALF2 = N - HALF1
LEFT_SLOTS = 6
TAIL_BASE = MAIN_ROWS + LEFT_SLOTS * SLOT
TAIL_REM_BASE = TAIL_BASE + CHUNK
TAIL_REM = HALF2 - TAIL_REM_BASE
NSLOTS_1 = NW * MAIN_ITERS
NSLOTS_2 = NW * MAIN_ITERS + LEFT_SLOTS + 1
PIDS_PER_TILE = B // NW


def _make_sc_half(with_leftover, with_targets):
    mesh = plsc.VectorSubcoreMesh(core_axis_name="c", subcore_axis_name="s")

    out_type = [
        jax.ShapeDtypeStruct((NC, CP, F), jnp.float32),
        jax.ShapeDtypeStruct((NC, CP, 16), jnp.float32),
    ]
    if with_targets:
        out_type.append(jax.ShapeDtypeStruct((B,), jnp.int32))

    @functools.partial(
        pl.kernel,
        mesh=mesh,
        compiler_params=pltpu.CompilerParams(needs_layout_passes=False),
        out_type=out_type,
        scratch_types=[
            pltpu.VMEM((SLOT, F), jnp.float32),
            pltpu.VMEM((SLOT, F), jnp.float32),
            pltpu.VMEM((SLOT // CHUNK, CHUNK), jnp.int32),
            pltpu.VMEM((SLOT // CHUNK, CHUNK), jnp.int32),
            pltpu.VMEM((TAIL_REM, F), jnp.float32),
            pltpu.VMEM((1, TAIL_REM), jnp.int32),
            pltpu.VMEM((CHUNK, 16), jnp.float32),
            pltpu.VMEM((16, 16), jnp.float32),
            pltpu.VMEM((PIDS_PER_TILE,), jnp.int32),
            pltpu.VMEM((PIDS_PER_TILE,), jnp.int32),
            pltpu.VMEM((16, F), jnp.float32),
            pltpu.VMEM_SHARED((CP, F), jnp.float32),
            pltpu.VMEM_SHARED((CP, 16), jnp.float32),
            pltpu.SemaphoreType.DMA,
            pltpu.SemaphoreType.DMA,
        ],
    )
    def sc_kernel(feat_hbm, lab3_hbm, lab1_hbm, pid_hbm, *outs_and_scratch):
        if with_targets:
            (acc_out, cnt_out, tgt_out,
             feat_buf0, feat_buf1, lab_buf0, lab_buf1, feat_tail, lab_tail,
             ones_buf, zcnt, pid_buf, tgt_buf, zrow, acc_sh, cnt_sh,
             stage_sem0, stage_sem1) = outs_and_scratch
        else:
            tgt_out = None
            (acc_out, cnt_out,
             feat_buf0, feat_buf1, lab_buf0, lab_buf1, feat_tail, lab_tail,
             ones_buf, zcnt, pid_buf, tgt_buf, zrow, acc_sh, cnt_sh,
             stage_sem0, stage_sem1) = outs_and_scratch
        cid = lax.axis_index("c")
        sid = lax.axis_index("s")
        wid = sid * NC + cid

        feat_bufs = (feat_buf0, feat_buf1)
        lab_bufs = (lab_buf0, lab_buf1)
        stage_sems = (stage_sem0, stage_sem1)

        zero16 = jnp.zeros((16,), jnp.float32)
        one16 = jnp.full((16,), 1.0, jnp.float32)
        for r in range(16):
            for q in range(F // 16):
                zrow[r, pl.ds(q * 16, 16)] = zero16
            zcnt[r, pl.ds(0, 16)] = zero16
        for r in range(CHUNK):
            ones_buf[r, pl.ds(0, 16)] = one16

        def fire_stage(j, b):
            g = wid + NW * j
            h1 = pltpu.async_copy(lab3_hbm.at[g], lab_bufs[b], stage_sems[b])
            h2 = pltpu.async_copy(feat_hbm.at[pl.ds(g * SLOT, SLOT)],
                                  feat_bufs[b], stage_sems[b])
            return [h1, h2]

        rows_per_tile = CP // NS
        for blk in range(rows_per_tile // 16):
            base = sid * rows_per_tile + blk * 16
            pltpu.sync_copy(zrow, acc_sh.at[pl.ds(base, 16)])
            pltpu.sync_copy(zcnt, cnt_sh.at[pl.ds(base, 16)])

        if with_targets:
            pltpu.sync_copy(pid_hbm.at[pl.ds(wid * PIDS_PER_TILE, PIDS_PER_TILE)],
                            pid_buf)
            pltpu.sync_copy(lab1_hbm.at[pid_buf], tgt_buf)
            pltpu.sync_copy(tgt_buf,
                            tgt_out.at[pl.ds(wid * PIDS_PER_TILE, PIDS_PER_TILE)])

        plsc.subcore_barrier()

        stageh = [None, None]
        stageh[0] = fire_stage(0, 0)
        for j in range(MAIN_ITERS):
            b = j & 1
            for h in stageh[b]:
                h.wait()
            if j + 1 < MAIN_ITERS:
                stageh[1 - b] = fire_stage(j + 1, 1 - b)
            for k in range(SLOT // CHUNK):
                pltpu.sync_copy(feat_bufs[b].at[pl.ds(k * CHUNK, CHUNK)],
                                acc_sh.at[lab_bufs[b].at[k]], add=True)
                pltpu.sync_copy(ones_buf, cnt_sh.at[lab_bufs[b].at[k]], add=True)

        if with_leftover:
            @pl.when(wid < LEFT_SLOTS)
            def _left():
                g = NW * MAIN_ITERS + wid
                pltpu.sync_copy(lab3_hbm.at[g], lab_buf0)
                pltpu.sync_copy(feat_hbm.at[pl.ds(g * SLOT, SLOT)], feat_buf0)
                for k in range(SLOT // CHUNK):
                    pltpu.sync_copy(feat_buf0.at[pl.ds(k * CHUNK, CHUNK)],
                                    acc_sh.at[lab_buf0.at[k]], add=True)
                    pltpu.sync_copy(ones_buf, cnt_sh.at[lab_buf0.at[k]], add=True)

            @pl.when(wid == LEFT_SLOTS)
            def _tail():
                g = NW * MAIN_ITERS + LEFT_SLOTS
                pltpu.sync_copy(lab3_hbm.at[g], lab_buf0)
                pltpu.sync_copy(feat_hbm.at[pl.ds(TAIL_BASE, CHUNK)],
                                feat_buf0.at[pl.ds(0, CHUNK)])
                pltpu.sync_copy(feat_buf0.at[pl.ds(0, CHUNK)],
                                acc_sh.at[lab_buf0.at[0]], add=True)
                pltpu.sync_copy(ones_buf, cnt_sh.at[lab_buf0.at[0]], add=True)
                pltpu.sync_copy(lab1_hbm.at[pl.ds(TAIL_REM_BASE, TAIL_REM)],
                                lab_tail.at[0])
                pltpu.sync_copy(feat_hbm.at[pl.ds(TAIL_REM_BASE, TAIL_REM)],
                                feat_tail)
                pltpu.sync_copy(feat_tail, acc_sh.at[lab_tail.at[0]], add=True)
                pltpu.sync_copy(ones_buf.at[pl.ds(0, TAIL_REM)],
                                cnt_sh.at[lab_tail.at[0]], add=True)

        plsc.subcore_barrier()

        @pl.when(sid == 0)
        def _publish():
            pltpu.sync_copy(acc_sh, acc_out.at[cid])
            pltpu.sync_copy(cnt_sh, cnt_out.at[cid])

    return sc_kernel


_sc_half1 = _make_sc_half(with_leftover=False, with_targets=True)
_sc_half2 = _make_sc_half(with_leftover=True, with_targets=False)


def _tc_loss(inputs, acc1, cnt1, acc2, cnt2, tgt):
    def body(x_ref, acc1_ref, cnt1_ref, acc2_ref, cnt2_ref, tgt_ref, out_ref):
        x = x_ref[...]
        cs = (acc1_ref[0] + acc1_ref[1]) + (acc2_ref[0] + acc2_ref[1])
        counts = (cnt1_ref[0, :, 0:1] + cnt1_ref[1, :, 0:1]
                  + cnt2_ref[0, :, 0:1] + cnt2_ref[1, :, 0:1])
        sim = lax.dot_general(cs, x, (((1,), (1,)), ((), ())),
                              preferred_element_type=jnp.float32)
        valid = (counts > 0.0) & (
            lax.broadcasted_iota(jnp.int32, (CP, 1), 0) < C)
        denom = jnp.where(counts > 0.0, counts, 1.0) * TEMP
        sim = sim / denom
        exps = jnp.exp(sim) * valid.astype(jnp.float32)
        sums = jnp.sum(exps, axis=0, keepdims=True) + 1e-6
        msim = exps / sums
        logp = jnp.log(msim + 1e-6)
        tgt_row = jnp.reshape(tgt_ref[...], (1, B))
        onehot = lax.broadcasted_iota(jnp.int32, (CP, B), 0) == tgt_row
        chosen = jnp.sum(jnp.where(onehot, logp, 0.0), axis=0)
        loss = -jnp.sum(chosen) / float(B)
        out_ref[...] = jnp.reshape(loss, (1, 1))

    out = pl.pallas_call(
        body,
        out_shape=jax.ShapeDtypeStruct((1, 1), jnp.float32),
    )(inputs, acc1, cnt1, acc2, cnt2, tgt)
    return out[0, 0]


def kernel(inputs, gt_labels, features, labels):
    pids = gt_labels[:, :, -1].reshape(-1)
    feat1 = features[:HALF1]
    feat2 = features[HALF1:]
    lab1_3d = labels[:HALF1].reshape(NSLOTS_1, SLOT // CHUNK, CHUNK)
    lab2_1d = labels[HALF1:]
    lab2_3d = jnp.pad(lab2_1d, (0, NSLOTS_2 * SLOT - HALF2)).reshape(
        NSLOTS_2, SLOT // CHUNK, CHUNK)
    acc1, cnt1, tgt = _sc_half1(feat1, lab1_3d, labels, pids)
    acc2, cnt2 = _sc_half2(feat2, lab2_3d, lab2_1d, pids)
    return _tc_loss(inputs, acc1, cnt1, acc2, cnt2, tgt)


# speedup vs baseline: 1.1626x; 1.1626x over previous
import functools

import jax
import jax.numpy as jnp
from jax import lax
from jax.experimental import pallas as pl
from jax.experimental.pallas import tpu as pltpu
from jax.experimental.pallas import tpu_sc as plsc

N = 100000
F = 64
C = 751
CP = 768
B = 1024
TEMP = 0.05

NC = 2
NS = 16
NW = NC * NS

CHUNK = 128
SLOT = 256
MAIN_ITERS = 12
MAIN_SLOTS = NW * MAIN_ITERS
LEFT_SLOTS = 6
TAIL_BASE = (MAIN_SLOTS + LEFT_SLOTS) * SLOT
TAIL_REM_BASE = TAIL_BASE + CHUNK
TAIL_REM = N - TAIL_REM_BASE
NSLOTS3 = MAIN_SLOTS + LEFT_SLOTS + 1
PIDS_PER_TILE = B // NW


def _sc_segment_sum(features, labels3d, labels1d, pids):
    mesh = plsc.VectorSubcoreMesh(core_axis_name="c", subcore_axis_name="s")

    @functools.partial(
        pl.kernel,
        mesh=mesh,
        compiler_params=pltpu.CompilerParams(needs_layout_passes=False),
        out_type=[
            jax.ShapeDtypeStruct((NC, CP, F), jnp.float32),
            jax.ShapeDtypeStruct((NC, CP, 16), jnp.float32),
            jax.ShapeDtypeStruct((B,), jnp.int32),
        ],
        scratch_types=[
            pltpu.VMEM((SLOT, F), jnp.float32),
            pltpu.VMEM((SLOT, F), jnp.float32),
            pltpu.VMEM((SLOT // CHUNK, CHUNK), jnp.int32),
            pltpu.VMEM((SLOT // CHUNK, CHUNK), jnp.int32),
            pltpu.VMEM((TAIL_REM, F), jnp.float32),
            pltpu.VMEM((1, TAIL_REM), jnp.int32),
            pltpu.VMEM((CHUNK, 16), jnp.float32),
            pltpu.VMEM((16, 16), jnp.float32),
            pltpu.VMEM((PIDS_PER_TILE,), jnp.int32),
            pltpu.VMEM((PIDS_PER_TILE,), jnp.int32),
            pltpu.VMEM((16, F), jnp.float32),
            pltpu.VMEM_SHARED((CP, F), jnp.float32),
            pltpu.VMEM_SHARED((CP, 16), jnp.float32),
            pltpu.SemaphoreType.DMA,
            pltpu.SemaphoreType.DMA,
            pltpu.SemaphoreType.DMA,
            pltpu.SemaphoreType.DMA,
        ],
    )
    def sc_kernel(feat_hbm, lab3_hbm, lab1_hbm, pid_hbm, acc_out, cnt_out, tgt_out,
                  feat_buf0, feat_buf1, lab_buf0, lab_buf1, feat_tail, lab_tail,
                  ones_buf, zcnt, pid_buf, tgt_buf, zrow, acc_sh, cnt_sh,
                  stage_sem0, stage_sem1, scat_sem0, scat_sem1):
        cid = lax.axis_index("c")
        sid = lax.axis_index("s")
        wid = sid * NC + cid

        feat_bufs = (feat_buf0, feat_buf1)
        lab_bufs = (lab_buf0, lab_buf1)
        stage_sems = (stage_sem0, stage_sem1)
        scat_sems = (scat_sem0, scat_sem1)

        zero16 = jnp.zeros((16,), jnp.float32)
        one16 = jnp.full((16,), 1.0, jnp.float32)
        for r in range(16):
            for q in range(F // 16):
                zrow[r, pl.ds(q * 16, 16)] = zero16
            zcnt[r, pl.ds(0, 16)] = zero16
        for r in range(CHUNK):
            ones_buf[r, pl.ds(0, 16)] = one16

        def fire_stage(j, b):
            g = wid + NW * j
            h1 = pltpu.async_copy(lab3_hbm.at[g], lab_bufs[b], stage_sems[b])
            h2 = pltpu.async_copy(feat_hbm.at[pl.ds(g * SLOT, SLOT)],
                                  feat_bufs[b], stage_sems[b])
            return [h1, h2]

        rows_per_tile = CP // NS
        for blk in range(rows_per_tile // 16):
            base = sid * rows_per_tile + blk * 16
            pltpu.sync_copy(zrow, acc_sh.at[pl.ds(base, 16)])
            pltpu.sync_copy(zcnt, cnt_sh.at[pl.ds(base, 16)])

        pltpu.sync_copy(pid_hbm.at[pl.ds(wid * PIDS_PER_TILE, PIDS_PER_TILE)], pid_buf)
        pltpu.sync_copy(lab1_hbm.at[pid_buf], tgt_buf)
        pltpu.sync_copy(tgt_buf, tgt_out.at[pl.ds(wid * PIDS_PER_TILE, PIDS_PER_TILE)])

        plsc.subcore_barrier()

        stageh = [None, None]
        scath = [None, None]
        stageh[0] = fire_stage(0, 0)
        for j in range(MAIN_ITERS):
            b = j & 1
            for h in stageh[b]:
                h.wait()
            if j + 1 < MAIN_ITERS:
                if scath[1 - b] is not None:
                    for h in scath[1 - b]:
                        h.wait()
                stageh[1 - b] = fire_stage(j + 1, 1 - b)
            hs = []
            for k in range(SLOT // CHUNK):
                hs.append(pltpu.async_copy(
                    feat_bufs[b].at[pl.ds(k * CHUNK, CHUNK)],
                    acc_sh.at[lab_bufs[b].at[k]], scat_sems[b], add=True))
                hs.append(pltpu.async_copy(
                    ones_buf, cnt_sh.at[lab_bufs[b].at[k]], scat_sems[b], add=True))
            scath[b] = hs
        for b2 in (0, 1):
            if scath[b2] is not None:
                for h in scath[b2]:
                    h.wait()

        @pl.when(wid < LEFT_SLOTS)
        def _left():
            g = MAIN_SLOTS + wid
            pltpu.sync_copy(lab3_hbm.at[g], lab_buf0)
            pltpu.sync_copy(feat_hbm.at[pl.ds(g * SLOT, SLOT)], feat_buf0)
            for k in range(SLOT // CHUNK):
                pltpu.sync_copy(feat_buf0.at[pl.ds(k * CHUNK, CHUNK)],
                                acc_sh.at[lab_buf0.at[k]], add=True)
                pltpu.sync_copy(ones_buf, cnt_sh.at[lab_buf0.at[k]], add=True)

        @pl.when(wid == LEFT_SLOTS)
        def _tail():
            g = MAIN_SLOTS + LEFT_SLOTS
            pltpu.sync_copy(lab3_hbm.at[g], lab_buf0)
            pltpu.sync_copy(feat_hbm.at[pl.ds(TAIL_BASE, CHUNK)],
                            feat_buf0.at[pl.ds(0, CHUNK)])
            pltpu.sync_copy(feat_buf0.at[pl.ds(0, CHUNK)],
                            acc_sh.at[lab_buf0.at[0]], add=True)
            pltpu.sync_copy(ones_buf, cnt_sh.at[lab_buf0.at[0]], add=True)
            pltpu.sync_copy(lab1_hbm.at[pl.ds(TAIL_REM_BASE, TAIL_REM)], lab_tail.at[0])
            pltpu.sync_copy(feat_hbm.at[pl.ds(TAIL_REM_BASE, TAIL_REM)], feat_tail)
            pltpu.sync_copy(feat_tail, acc_sh.at[lab_tail.at[0]], add=True)
            pltpu.sync_copy(ones_buf.at[pl.ds(0, TAIL_REM)], cnt_sh.at[lab_tail.at[0]], add=True)

        plsc.subcore_barrier()

        @pl.when(sid == 0)
        def _publish():
            pltpu.sync_copy(acc_sh, acc_out.at[cid])
            pltpu.sync_copy(cnt_sh, cnt_out.at[cid])

    return sc_kernel(features, labels3d, labels1d, pids)


def _tc_loss(inputs, acc, cnt, tgt):
    def body(x_ref, acc_ref, cnt_ref, tgt_ref, out_ref):
        x = x_ref[...]
        cs = acc_ref[0] + acc_ref[1]
        counts = cnt_ref[0, :, 0:1] + cnt_ref[1, :, 0:1]
        sim = lax.dot_general(cs, x, (((1,), (1,)), ((), ())),
                              preferred_element_type=jnp.float32)
        valid = (counts > 0.0) & (
            lax.broadcasted_iota(jnp.int32, (CP, 1), 0) < C)
        denom = jnp.where(counts > 0.0, counts, 1.0) * TEMP
        sim = sim / denom
        exps = jnp.exp(sim) * valid.astype(jnp.float32)
        sums = jnp.sum(exps, axis=0, keepdims=True) + 1e-6
        msim = exps / sums
        logp = jnp.log(msim + 1e-6)
        tgt_row = jnp.reshape(tgt_ref[...], (1, B))
        onehot = lax.broadcasted_iota(jnp.int32, (CP, B), 0) == tgt_row
        chosen = jnp.sum(jnp.where(onehot, logp, 0.0), axis=0)
        loss = -jnp.sum(chosen) / float(B)
        out_ref[...] = jnp.reshape(loss, (1, 1))

    out = pl.pallas_call(
        body,
        out_shape=jax.ShapeDtypeStruct((1, 1), jnp.float32),
    )(inputs, acc, cnt, tgt)
    return out[0, 0]


def kernel(inputs, gt_labels, features, labels):
    pids = gt_labels[:, :, -1].reshape(-1)
    labels3d = jnp.pad(labels, (0, NSLOTS3 * SLOT - N)).reshape(
        NSLOTS3, SLOT // CHUNK, CHUNK)
    acc, cnt, tgt = _sc_segment_sum(features, labels3d, labels, pids)
    return _tc_loss(inputs, acc, cnt, tgt)
